# baseline (speedup 1.0000x reference)
import jax
import jax.numpy as jnp
import numpy as np
from jax.experimental import pallas as pl
from jax.experimental.pallas import tpu as pltpu

_NCHUNK = 8


def _body(maskr, epsT, pack_h, feat_h, adj_hbm,
          act_o, val_o,
          pack, feat, a_vmem, sem_in, sem_adj):
    f32 = jnp.float32
    n = a_vmem.shape[0]
    rows = n // _NCHUNK

    def in_cp(i):
        src, dst = ((pack_h, pack), (feat_h, feat))[i]
        return pltpu.make_async_copy(src, dst, sem_in.at[i])

    def adj_cp(k):
        return pltpu.make_async_copy(
            adj_hbm.at[pl.ds(k * rows, rows), :],
            a_vmem.at[pl.ds(k * rows, rows), :],
            sem_adj.at[k])

    in_cp(0).start()
    in_cp(1).start()
    for k in range(_NCHUNK):
        adj_cp(k).start()
    in_cp(0).wait()
    in_cp(1).wait()

    We1 = pack[0:256, :]
    We2 = pack[256:288, :]
    Wg = pack[288:320, :]
    Wgd = pack[320:352, :]
    Wp1a = pack[352:384, :]
    Wp1b = pack[384:416, :]
    Wp2 = pack[416:448, :]
    Wh = pack[448:480, :]
    be1 = pack[480:481, :]
    be2 = pack[481:482, :]
    bg = pack[482:483, :]
    bgd = pack[483:484, :]
    bp1 = pack[484:485, :]
    bp2 = pack[485:486, :]
    bh = pack[486:487, 0:6]

    X1 = jax.nn.relu(jnp.dot(feat[:], We1, preferred_element_type=f32) + be1)
    Xn = jax.nn.relu(jnp.dot(X1, We2, preferred_element_type=f32) + be2)
    Xln = jnp.dot(Xn, Wg, preferred_element_type=f32)
    XT = Xn.T
    XlT = Xln.T
    deg = jnp.full((1, n), 1.0, f32)
    for k in range(_NCHUNK):
        adj_cp(k).wait()
        deg = deg + jnp.sum(a_vmem[pl.ds(k * rows, rows), :], axis=0, keepdims=True)
    dinv = jnp.where(deg > 0, 1.0 / jnp.sqrt(deg), 0.0)
    ST = XlT * dinv
    Y0T = jax.lax.dot_general(ST, a_vmem[:], (((1,), (0,)), ((), ())),
                              precision=jax.lax.Precision.DEFAULT,
                              preferred_element_type=f32)
    YT = Y0T * dinv + XlT * (dinv * dinv)
    XgT = jax.nn.relu(YT + bg.T)
    Xg2T = jax.nn.relu(jnp.dot(Wgd.T, XgT, preferred_element_type=f32) + bgd.T)
    XpT = jax.nn.relu(jnp.dot(Wp1a.T, Xg2T, preferred_element_type=f32)
                      + jnp.dot(Wp1b.T, XT, preferred_element_type=f32)
                      + bp1.T)
    XpT = jax.nn.relu(jnp.dot(Wp2.T, XpT, preferred_element_type=f32) + bp2.T)
    HT = jnp.dot(Wh[:, 0:6].T, XpT, preferred_element_type=f32) + bh.T
    val_o[:] = HT[0:1, :].T
    muT = jnp.tanh(HT[1:3, :])
    zT = jnp.tanh(HT[3:6, :])
    sigT = jnp.concatenate([jnp.exp(-zT[0:1, :]), jnp.exp(-zT[2:3, :])], axis=0)
    act = jnp.clip(muT + epsT[:] * sigT, -1.0, 1.0) * maskr[:]
    act_o[:] = act.T


def kernel(features, adjacency, mask, We1, be1, We2, be2, Wg, bg, Wgd, bgd,
           Wp1, bp1, Wp2, bp2, Wv, bv, Wmu, bmu, WL, bL):
    n, fdim = features.shape
    A = Wmu.shape[1]
    epsT = jax.random.normal(jax.random.key(42), (n, A), jnp.float32).T
    Wh = jnp.concatenate([Wv, Wmu, WL], axis=1)
    Whp = jnp.pad(Wh, ((0, 0), (0, 26)))
    bh = jnp.concatenate([bv, bmu, bL], axis=0)
    brows = jnp.stack([be1, be2, bg, bgd, bp1, bp2,
                       jnp.pad(bh, (0, 26)), jnp.zeros((32,), jnp.float32)])
    pack = jnp.concatenate(
        [We1, We2, Wg, Wgd, Wp1[0:32], Wp1[32:64], Wp2, Whp, brows], axis=0)
    hbm = pl.BlockSpec(memory_space=pltpu.MemorySpace.HBM)
    vmem = pl.BlockSpec(memory_space=pltpu.MemorySpace.VMEM)
    act, val = pl.pallas_call(
        _body,
        in_specs=[vmem, vmem, hbm, hbm, hbm],
        out_shape=(
            jax.ShapeDtypeStruct((n, A), jnp.float32),
            jax.ShapeDtypeStruct((n, 1), jnp.float32),
        ),
        scratch_shapes=[
            pltpu.MemorySpace.VMEM(pack.shape, jnp.float32),
            pltpu.MemorySpace.VMEM(features.shape, jnp.float32),
            pltpu.MemorySpace.VMEM((n, n), jnp.float32),
            pltpu.SemaphoreType.DMA((2,)),
            pltpu.SemaphoreType.DMA((_NCHUNK,)),
        ],
    )(mask.reshape(1, n), epsT, pack, features, adjacency)
    return (act, val)


# speedup vs baseline: 1.1777x; 1.1777x over previous
import functools

import jax
import jax.numpy as jnp
import numpy as np
from jax.experimental import pallas as pl
from jax.experimental.pallas import tpu as pltpu

_NCHUNK = 16


def _body(maskr, epsT_h,
          feat_h, We1_h, be1_h, We2_h, be2_h, Wg_h, bg_h, Wgd_h, bgd_h,
          Wp1_h, bp1_h, Wp2_h, bp2_h, Wv_h, bv_h, Wmu_h, bmu_h, WL_h, bL_h,
          adj_hbm,
          act_o, val_o,
          epsT, feat, We1, be1, We2, be2, Wg, bg, Wgd, bgd,
          Wp1, bp1, Wp2, bp2, Wv, bv, Wmu, bmu, WL, bL,
          a_vmem, sem_in, sem_adj):
    f32 = jnp.float32
    n = a_vmem.shape[0]
    rows = n // _NCHUNK

    ins = [(epsT_h, epsT), (feat_h, feat), (We1_h, We1), (be1_h, be1), (We2_h, We2),
           (be2_h, be2), (Wg_h, Wg), (bg_h, bg), (Wgd_h, Wgd), (bgd_h, bgd),
           (Wp1_h, Wp1), (bp1_h, bp1), (Wp2_h, Wp2), (bp2_h, bp2),
           (Wv_h, Wv), (bv_h, bv), (Wmu_h, Wmu), (bmu_h, bmu),
           (WL_h, WL), (bL_h, bL)]

    def in_cp(i):
        return pltpu.make_async_copy(ins[i][0], ins[i][1], sem_in.at[i])

    def adj_cp(k):
        return pltpu.make_async_copy(
            adj_hbm.at[pl.ds(k * rows, rows), :],
            a_vmem.at[pl.ds(k * rows, rows), :],
            sem_adj.at[k])

    for i in range(len(ins)):
        in_cp(i).start()
    for k in range(_NCHUNK):
        adj_cp(k).start()

    for i in range(len(ins)):
        in_cp(i).wait()
    X1 = jax.nn.relu(jnp.dot(feat[:], We1[:], preferred_element_type=f32) + be1[:])
    Xn = jax.nn.relu(jnp.dot(X1, We2[:], preferred_element_type=f32) + be2[:])
    Xln = jnp.dot(Xn, Wg[:], preferred_element_type=f32)
    XT = Xn.T
    XlT = Xln.T
    deg = jnp.full((1, n), 1.0, f32)
    for k in range(_NCHUNK):
        adj_cp(k).wait()
        deg = deg + jnp.sum(a_vmem[pl.ds(k * rows, rows), :], axis=0, keepdims=True)
    dinv = jnp.where(deg > 0, 1.0 / jnp.sqrt(deg), 0.0)
    ST = XlT * dinv
    Y0T = jax.lax.dot_general(ST, a_vmem[:], (((1,), (0,)), ((), ())),
                              precision=jax.lax.Precision.DEFAULT,
                              preferred_element_type=f32)
    YT = Y0T * dinv + XlT * (dinv * dinv)
    XgT = jax.nn.relu(YT + bg[:].T)
    Xg2T = jax.nn.relu(jnp.dot(Wgd[:].T, XgT, preferred_element_type=f32) + bgd[:].T)
    XpT = jax.nn.relu(jnp.dot(Wp1[0:32, :].T, Xg2T, preferred_element_type=f32)
                      + jnp.dot(Wp1[32:64, :].T, XT, preferred_element_type=f32)
                      + bp1[:].T)
    XpT = jax.nn.relu(jnp.dot(Wp2[:].T, XpT, preferred_element_type=f32) + bp2[:].T)
    Wh = jnp.concatenate([Wv[:], Wmu[:], WL[:]], axis=1)
    bh = jnp.concatenate([bv[:], bmu[:], bL[:]], axis=1)
    HT = jnp.dot(Wh.T, XpT, preferred_element_type=f32) + bh.T
    val_o[:] = HT[0:1, :].T
    muT = jnp.tanh(HT[1:3, :])
    zT = jnp.tanh(HT[3:6, :])
    sigT = jnp.concatenate([jnp.exp(-zT[0:1, :]), jnp.exp(-zT[2:3, :])], axis=0)
    act = jnp.clip(muT + epsT[:] * sigT, -1.0, 1.0) * maskr[:]
    act_o[:] = act.T


def kernel(features, adjacency, mask, We1, be1, We2, be2, Wg, bg, Wgd, bgd,
           Wp1, bp1, Wp2, bp2, Wv, bv, Wmu, bmu, WL, bL):
    n, fdim = features.shape
    A = Wmu.shape[1]
    epsT = jax.random.normal(jax.random.key(42), (n, A), jnp.float32).T
    hbm = pl.BlockSpec(memory_space=pltpu.MemorySpace.HBM)
    vmem = pl.BlockSpec(memory_space=pltpu.MemorySpace.VMEM)
    args = (
        mask.reshape(1, n),
        epsT,
        features,
        We1, be1.reshape(1, -1),
        We2, be2.reshape(1, -1),
        Wg, bg.reshape(1, -1),
        Wgd, bgd.reshape(1, -1),
        Wp1, bp1.reshape(1, -1),
        Wp2, bp2.reshape(1, -1),
        Wv, bv.reshape(1, -1),
        Wmu, bmu.reshape(1, -1),
        WL, bL.reshape(1, -1),
        adjacency,
    )
    scratch = [pltpu.MemorySpace.VMEM(a.shape, jnp.float32) for a in args[1:-1]]
    act, val = pl.pallas_call(
        _body,
        in_specs=[vmem] + [hbm] * (len(args) - 1),
        out_shape=(
            jax.ShapeDtypeStruct((n, A), jnp.float32),
            jax.ShapeDtypeStruct((n, 1), jnp.float32),
        ),
        scratch_shapes=scratch + [
            pltpu.MemorySpace.VMEM((n, n), jnp.float32),
            pltpu.SemaphoreType.DMA((len(args) - 2,)),
            pltpu.SemaphoreType.DMA((_NCHUNK,)),
        ],
    )(*args)
    return (act, val)
